# baseline (speedup 1.0000x reference)
import jax
import jax.numpy as jnp
from jax import lax
from jax.experimental import pallas as pl
from jax.experimental.pallas import tpu as pltpu
from jax.experimental.pallas import tpu_sc as plsc

B, L, D = 8, 512, 256
K = 16
HD = D // 2
HT = L // 2
ZROW = L
NTV = HT // 16
NC = HD // 16
HW = HD // 2
NW = HW // 16


def _body(inp, heads, scal, out,
          inp_v, heads_v, scal_v, depth_v, apd_v, cidx_v, mrg_v, pmax_v,
          obuf, lsem, wsem):
    wid = lax.axis_index("s") * 2 + lax.axis_index("c")
    b = wid // 4
    th = (wid % 4) // 2
    chf = wid % 2
    tbase = th * HT
    gbase = b * L + tbase
    iota = lax.iota(jnp.int32, 16)

    lh = pltpu.async_copy(
        inp.at[pl.ds(chf * (B * L) + b * L, L)],
        inp_v.at[pl.ds(0, L)], lsem)
    pltpu.sync_copy(heads.at[pl.ds(b * L, L)], heads_v)
    pltpu.sync_copy(scal, scal_v)

    p_vec = plsc.load_gather(scal_v, [jnp.full((16,), b, jnp.int32)])
    len_vec = plsc.load_gather(scal_v, [jnp.full((16,), b + 8, jnp.int32)])

    @plsc.parallel_loop(0, L // 16, unroll=2)
    def depth_body(tv):
        ids = iota + tv * 16
        cur = ids
        d = jnp.zeros((16,), jnp.int32)
        for _k in range(1, K):
            nxt = plsc.load_gather(heads_v, [cur])
            d = d + jnp.where(nxt != cur, 1, 0)
            cur = nxt
        depth_v[pl.ds(tv * 16, 16)] = d

    cur = p_vec
    cp = jnp.where(iota == 0, cur, 0)
    for k in range(1, K):
        cur = plsc.load_gather(heads_v, [cur])
        cp = jnp.where(iota == k, cur, cp)
    depth_p_vec = plsc.load_gather(depth_v, [p_vec])
    dvals = plsc.load_gather(depth_v, [cp])

    @plsc.parallel_loop(0, L // 16, unroll=2)
    def apd_init(tv):
        apd_v[pl.ds(tv * 16, 16)] = jnp.full((16,), -1, jnp.int32)
    plsc.store_scatter(apd_v, [cp], dvals)

    @plsc.parallel_loop(0, NTV, unroll=2)
    def tok_idx_body(tv):
        ids = iota + tbase + tv * 16
        cur = ids
        lca = jnp.full((16,), -1, jnp.int32)
        raws = []
        for k in range(K):
            av = plsc.load_gather(apd_v, [cur])
            lca = jnp.maximum(lca, av)
            raws.append(cur)
            if k < K - 1:
                cur = plsc.load_gather(heads_v, [cur])
        dmy = plsc.load_gather(depth_v, [ids])
        sl = dmy - lca
        sr = depth_p_vec - lca
        mr = jnp.clip(sr, 0, K - 1)
        valid = ids < len_vec
        for k in range(K):
            g = raws[k] if k == 0 else jnp.where(k <= sl, raws[k], ids)
            cidx_v[k, pl.ds(tv * 16, 16)] = jnp.where(valid, g, ZROW)
        mrg_v[pl.ds(tv * 16, 16)] = jnp.where(valid, mr, K)

    zi = jnp.zeros((16,), jnp.int32)
    for cc in range(NW):
        inp_v[ZROW, pl.ds(cc * 16, 16)] = zi

    def bmax(a, b):
        return plsc.bitcast(
            jnp.maximum(plsc.bitcast(a, jnp.bfloat16),
                        plsc.bitcast(b, jnp.bfloat16)), jnp.int32)

    lh.wait()

    for cc in range(NW):
        pmax_v[K, pl.ds(cc * 16, 16)] = zi
    for m in range(K):
        r = cp[m]
        for cc in range(NW):
            row = inp_v[r, pl.ds(cc * 16, 16)]
            if m > 0:
                row = bmax(row, pmax_v[m - 1, pl.ds(cc * 16, 16)])
            pmax_v[m, pl.ds(cc * 16, 16)] = row

    @plsc.parallel_loop(0, NTV)
    def left_body(tv):
        idxv = [cidx_v[k, pl.ds(tv * 16, 16)] for k in range(K)]
        for j in range(16):
            rs = [idxv[k][j] for k in range(K)]
            row = tv * 16 + j
            for cc in range(NW):
                g = [plsc.bitcast(inp_v[rs[k], pl.ds(cc * 16, 16)],
                                  jnp.bfloat16) for k in range(K)]
                while len(g) > 1:
                    g = [jnp.maximum(g[i], g[i + 1])
                         for i in range(0, len(g), 2)]
                obuf[row, pl.ds(cc * 16, 16)] = plsc.bitcast(g[0], jnp.int32)

    lwh = pltpu.async_copy(
        obuf, out.at[pl.ds(chf * (B * L) + gbase, HT)], wsem)
    lwh.wait()

    @plsc.parallel_loop(0, NTV)
    def right_body(tv):
        mrgv = mrg_v[pl.ds(tv * 16, 16)]
        for j in range(16):
            m = mrgv[j]
            row = tv * 16 + j
            for cc in range(NW):
                obuf[row, pl.ds(cc * 16, 16)] = pmax_v[m, pl.ds(cc * 16, 16)]

    pltpu.sync_copy(
        obuf, out.at[pl.ds((2 + chf) * (B * L) + gbase, HT)])


_call = pl.kernel(
    _body,
    out_type=jax.ShapeDtypeStruct((4 * B * L, HW), jnp.int32),
    mesh=plsc.VectorSubcoreMesh(core_axis_name="c", subcore_axis_name="s"),
    compiler_params=pltpu.CompilerParams(needs_layout_passes=False),
    scratch_types=[
        pltpu.VMEM((L + 8, HW), jnp.int32),
        pltpu.VMEM((L,), jnp.int32),
        pltpu.VMEM((16,), jnp.int32),
        pltpu.VMEM((L,), jnp.int32),
        pltpu.VMEM((L,), jnp.int32),
        pltpu.VMEM((K, HT), jnp.int32),
        pltpu.VMEM((HT,), jnp.int32),
        pltpu.VMEM((K + 1, HW), jnp.int32),
        pltpu.VMEM((HT, HW), jnp.int32),
        pltpu.SemaphoreType.DMA,
        pltpu.SemaphoreType.DMA,
    ],
)


def kernel(inputs, heads, predicates, lengths):
    xbf = inputs.reshape(B * L, D // 2, 2).astype(jnp.bfloat16)
    inp = jax.lax.bitcast_convert_type(xbf, jnp.int32)
    inp = inp.reshape(B * L, 2, HW).transpose(1, 0, 2).reshape(2 * B * L, HW)
    heads_f = heads.reshape(B * L).astype(jnp.int32)
    scal = jnp.concatenate(
        [predicates.astype(jnp.int32), lengths.astype(jnp.int32)])
    out = _call(inp, heads_f, scal)
    out = out.reshape(4, B * L, HW).transpose(1, 0, 2).reshape(B * L, D)
    obf = jax.lax.bitcast_convert_type(out, jnp.bfloat16)
    return obf.reshape(B, L, 2 * D).astype(jnp.float32)


# speedup vs baseline: 1.1076x; 1.1076x over previous
import jax
import jax.numpy as jnp
from jax import lax
from jax.experimental import pallas as pl
from jax.experimental.pallas import tpu as pltpu
from jax.experimental.pallas import tpu_sc as plsc

B, L, D = 8, 512, 256
K = 16
HD = D // 2
HT = L // 2
ZROW = L
NTV = HT // 16
NC = HD // 16


def _body(inp, heads, scal, out,
          inp_v, heads_v, scal_v, depth_v, apd_v, cidx_v, mrg_v, pmax_v,
          obuf, rbuf, lsem, wsem):
    wid = lax.axis_index("s") * 2 + lax.axis_index("c")
    b = wid // 4
    th = (wid % 4) // 2
    chf = wid % 2
    tbase = th * HT
    gbase = b * L + tbase
    iota = lax.iota(jnp.int32, 16)

    lh = pltpu.async_copy(
        inp.at[pl.ds(b * L, L), pl.ds(chf * HD, HD)],
        inp_v.at[pl.ds(0, L)], lsem)
    pltpu.sync_copy(heads.at[pl.ds(b * L, L)], heads_v)
    pltpu.sync_copy(scal, scal_v)

    p_vec = plsc.load_gather(scal_v, [jnp.full((16,), b, jnp.int32)])
    len_vec = plsc.load_gather(scal_v, [jnp.full((16,), b + 8, jnp.int32)])

    @plsc.parallel_loop(0, L // 16, unroll=2)
    def depth_body(tv):
        ids = iota + tv * 16
        cur = ids
        d = jnp.zeros((16,), jnp.int32)
        for _k in range(1, K):
            nxt = plsc.load_gather(heads_v, [cur])
            d = d + jnp.where(nxt != cur, 1, 0)
            cur = nxt
        depth_v[pl.ds(tv * 16, 16)] = d

    cur = p_vec
    cp = jnp.where(iota == 0, cur, 0)
    for k in range(1, K):
        cur = plsc.load_gather(heads_v, [cur])
        cp = jnp.where(iota == k, cur, cp)
    depth_p_vec = plsc.load_gather(depth_v, [p_vec])
    dvals = plsc.load_gather(depth_v, [cp])

    @plsc.parallel_loop(0, L // 16, unroll=2)
    def apd_init(tv):
        apd_v[pl.ds(tv * 16, 16)] = jnp.full((16,), -1, jnp.int32)
    plsc.store_scatter(apd_v, [cp], dvals)

    @plsc.parallel_loop(0, NTV, unroll=2)
    def tok_idx_body(tv):
        ids = iota + tbase + tv * 16
        cur = ids
        lca = jnp.full((16,), -1, jnp.int32)
        raws = []
        for k in range(K):
            av = plsc.load_gather(apd_v, [cur])
            lca = jnp.maximum(lca, av)
            raws.append(cur)
            if k < K - 1:
                cur = plsc.load_gather(heads_v, [cur])
        dmy = plsc.load_gather(depth_v, [ids])
        sl = dmy - lca
        sr = depth_p_vec - lca
        mr = jnp.clip(sr, 0, K - 1)
        valid = ids < len_vec
        for k in range(K):
            g = raws[k] if k == 0 else jnp.where(k <= sl, raws[k], ids)
            cidx_v[k, pl.ds(tv * 16, 16)] = jnp.where(valid, g, ZROW)
        mrg_v[pl.ds(tv * 16, 16)] = jnp.where(valid, mr, K)

    zf = jnp.zeros((16,), jnp.float32)
    for cc in range(NC):
        inp_v[ZROW, pl.ds(cc * 16, 16)] = zf

    lh.wait()

    for cc in range(NC):
        pmax_v[K, pl.ds(cc * 16, 16)] = zf
    for m in range(K):
        r = cp[m]
        for cc in range(NC):
            row = inp_v[r, pl.ds(cc * 16, 16)]
            if m > 0:
                row = jnp.maximum(row, pmax_v[m - 1, pl.ds(cc * 16, 16)])
            pmax_v[m, pl.ds(cc * 16, 16)] = row

    rwh = []
    for half in range(2):
        hb = half * (NTV // 2)
        if half == 1:
            rwh[0].wait()

        @plsc.parallel_loop(hb, hb + NTV // 2)
        def right_body(tv, half=half):
            mrgv = mrg_v[pl.ds(tv * 16, 16)]
            for j in range(16):
                m = mrgv[j]
                row = (tv - (half * (NTV // 2))) * 16 + j
                for cc in range(NC):
                    rbuf[row, pl.ds(cc * 16, 16)] = \
                        pmax_v[m, pl.ds(cc * 16, 16)]

        rwh.append(pltpu.async_copy(
            rbuf,
            out.at[pl.ds(gbase + half * (HT // 2), HT // 2),
                   pl.ds(D + chf * HD, HD)], wsem))

    @plsc.parallel_loop(0, NTV)
    def left_body(tv):
        idxv = [cidx_v[k, pl.ds(tv * 16, 16)] for k in range(K)]
        for j in range(16):
            rs = [idxv[k][j] for k in range(K)]
            row = tv * 16 + j
            for cc in range(NC):
                g = [inp_v[rs[k], pl.ds(cc * 16, 16)] for k in range(K)]
                while len(g) > 1:
                    g = [jnp.maximum(g[i], g[i + 1])
                         for i in range(0, len(g), 2)]
                obuf[row, pl.ds(cc * 16, 16)] = g[0]

    pltpu.sync_copy(
        obuf, out.at[pl.ds(gbase, HT), pl.ds(chf * HD, HD)])
    rwh[1].wait()


_call = pl.kernel(
    _body,
    out_type=jax.ShapeDtypeStruct((B * L, 2 * D), jnp.float32),
    mesh=plsc.VectorSubcoreMesh(core_axis_name="c", subcore_axis_name="s"),
    compiler_params=pltpu.CompilerParams(needs_layout_passes=False),
    scratch_types=[
        pltpu.VMEM((L + 8, HD), jnp.float32),
        pltpu.VMEM((L,), jnp.int32),
        pltpu.VMEM((16,), jnp.int32),
        pltpu.VMEM((L,), jnp.int32),
        pltpu.VMEM((L,), jnp.int32),
        pltpu.VMEM((K, HT), jnp.int32),
        pltpu.VMEM((HT,), jnp.int32),
        pltpu.VMEM((K + 1, HD), jnp.float32),
        pltpu.VMEM((HT, HD), jnp.float32),
        pltpu.VMEM((HT // 2, HD), jnp.float32),
        pltpu.SemaphoreType.DMA,
        pltpu.SemaphoreType.DMA,
    ],
)


def kernel(inputs, heads, predicates, lengths):
    inp = inputs.reshape(B * L, D)
    heads_f = heads.reshape(B * L).astype(jnp.int32)
    scal = jnp.concatenate(
        [predicates.astype(jnp.int32), lengths.astype(jnp.int32)])
    out = _call(inp, heads_f, scal)
    return out.reshape(B, L, 2 * D)


# speedup vs baseline: 1.1201x; 1.0113x over previous
import jax
import jax.numpy as jnp
from jax import lax
from jax.experimental import pallas as pl
from jax.experimental.pallas import tpu as pltpu
from jax.experimental.pallas import tpu_sc as plsc

B, L, D = 8, 512, 256
K = 16
HD = D // 2
HT = L // 2
ZROW = L
NTV = HT // 16
NC = HD // 16


def _body(inp, heads, scal, out,
          inp_v, heads_v, scal_v, depth_v, apd_v, cidx_v, mrg_v, pmax_v,
          obuf, lsem, wsem):
    wid = lax.axis_index("s") * 2 + lax.axis_index("c")
    b = wid // 4
    th = (wid % 4) // 2
    chf = wid % 2
    tbase = th * HT
    gbase = b * L + tbase
    iota = lax.iota(jnp.int32, 16)

    lh = pltpu.async_copy(
        inp.at[pl.ds(b * L, L), pl.ds(chf * HD, HD)],
        inp_v.at[pl.ds(0, L)], lsem)
    pltpu.sync_copy(heads.at[pl.ds(b * L, L)], heads_v)
    pltpu.sync_copy(scal, scal_v)

    p_vec = plsc.load_gather(scal_v, [jnp.full((16,), b, jnp.int32)])
    len_vec = plsc.load_gather(scal_v, [jnp.full((16,), b + 8, jnp.int32)])

    @plsc.parallel_loop(0, L // 16, unroll=2)
    def depth_body(tv):
        ids = iota + tv * 16
        cur = ids
        d = jnp.zeros((16,), jnp.int32)
        for _k in range(1, K):
            nxt = plsc.load_gather(heads_v, [cur])
            d = d + jnp.where(nxt != cur, 1, 0)
            cur = nxt
        depth_v[pl.ds(tv * 16, 16)] = d

    cur = p_vec
    cp = jnp.where(iota == 0, cur, 0)
    for k in range(1, K):
        cur = plsc.load_gather(heads_v, [cur])
        cp = jnp.where(iota == k, cur, cp)
    depth_p_vec = plsc.load_gather(depth_v, [p_vec])
    dvals = plsc.load_gather(depth_v, [cp])

    @plsc.parallel_loop(0, L // 16, unroll=2)
    def apd_init(tv):
        apd_v[pl.ds(tv * 16, 16)] = jnp.full((16,), -1, jnp.int32)
    plsc.store_scatter(apd_v, [cp], dvals)

    @plsc.parallel_loop(0, NTV, unroll=2)
    def tok_idx_body(tv):
        ids = iota + tbase + tv * 16
        cur = ids
        lca = jnp.full((16,), -1, jnp.int32)
        raws = []
        for k in range(K):
            av = plsc.load_gather(apd_v, [cur])
            lca = jnp.maximum(lca, av)
            raws.append(cur)
            if k < K - 1:
                cur = plsc.load_gather(heads_v, [cur])
        dmy = plsc.load_gather(depth_v, [ids])
        sl = dmy - lca
        sr = depth_p_vec - lca
        mr = jnp.clip(sr, 0, K - 1)
        valid = ids < len_vec
        for k in range(K):
            g = raws[k] if k == 0 else jnp.where(k <= sl, raws[k], ids)
            cidx_v[k, pl.ds(tv * 16, 16)] = jnp.where(valid, g, ZROW)
        mrg_v[pl.ds(tv * 16, 16)] = jnp.where(valid, mr, K)

    zf = jnp.zeros((16,), jnp.float32)
    for cc in range(NC):
        inp_v[ZROW, pl.ds(cc * 16, 16)] = zf

    lh.wait()

    for cc in range(NC):
        pmax_v[K, pl.ds(cc * 16, 16)] = zf
    for m in range(K):
        r = cp[m]
        for cc in range(NC):
            row = inp_v[r, pl.ds(cc * 16, 16)]
            if m > 0:
                row = jnp.maximum(row, pmax_v[m - 1, pl.ds(cc * 16, 16)])
            pmax_v[m, pl.ds(cc * 16, 16)] = row

    @plsc.parallel_loop(0, NTV)
    def left_body(tv):
        idxv = [cidx_v[k, pl.ds(tv * 16, 16)] for k in range(K)]
        for j in range(16):
            rs = [idxv[k][j] for k in range(K)]
            row = tv * 16 + j
            for cc in range(NC):
                g = [inp_v[rs[k], pl.ds(cc * 16, 16)] for k in range(K)]
                while len(g) > 1:
                    g = [jnp.maximum(g[i], g[i + 1])
                         for i in range(0, len(g), 2)]
                obuf[row, pl.ds(cc * 16, 16)] = g[0]

    lwh = pltpu.async_copy(
        obuf, out.at[pl.ds(gbase, HT), pl.ds(chf * HD, HD)], wsem)
    lwh.wait()

    @plsc.parallel_loop(0, NTV)
    def right_body(tv):
        mrgv = mrg_v[pl.ds(tv * 16, 16)]
        for j in range(16):
            m = mrgv[j]
            row = tv * 16 + j
            for cc in range(NC):
                obuf[row, pl.ds(cc * 16, 16)] = pmax_v[m, pl.ds(cc * 16, 16)]

    pltpu.sync_copy(
        obuf, out.at[pl.ds(gbase, HT), pl.ds(D + chf * HD, HD)])


_call = pl.kernel(
    _body,
    out_type=jax.ShapeDtypeStruct((B * L, 2 * D), jnp.float32),
    mesh=plsc.VectorSubcoreMesh(core_axis_name="c", subcore_axis_name="s"),
    compiler_params=pltpu.CompilerParams(needs_layout_passes=False),
    scratch_types=[
        pltpu.VMEM((L + 8, HD), jnp.float32),
        pltpu.VMEM((L,), jnp.int32),
        pltpu.VMEM((16,), jnp.int32),
        pltpu.VMEM((L,), jnp.int32),
        pltpu.VMEM((L,), jnp.int32),
        pltpu.VMEM((K, HT), jnp.int32),
        pltpu.VMEM((HT,), jnp.int32),
        pltpu.VMEM((K + 1, HD), jnp.float32),
        pltpu.VMEM((HT, HD), jnp.float32),
        pltpu.SemaphoreType.DMA,
        pltpu.SemaphoreType.DMA,
    ],
)


def kernel(inputs, heads, predicates, lengths):
    inp = inputs.reshape(B * L, D)
    heads_f = heads.reshape(B * L).astype(jnp.int32)
    scal = jnp.concatenate(
        [predicates.astype(jnp.int32), lengths.astype(jnp.int32)])
    out = _call(inp, heads_f, scal)
    return out.reshape(B, L, 2 * D)
